# baseline (speedup 1.0000x reference)
import functools

import jax
import jax.numpy as jnp
from jax import lax
from jax.experimental import pallas as pl
from jax.experimental.pallas import tpu as pltpu
from jax.experimental.pallas import tpu_sc as plsc

_D = 128
_N = 1024 * 200
_NW = 32
_PER_W = _N // _NW
_CHUNK = 200
_PAIR = 2 * _CHUNK
_NCHUNK = _PER_W // _CHUNK
_NLAP = _NCHUNK // 4

_mesh = plsc.VectorSubcoreMesh(core_axis_name="c", subcore_axis_name="s")


@functools.partial(
    pl.kernel,
    mesh=_mesh,
    out_type=jax.ShapeDtypeStruct((_N, _D), jnp.float32),
    scratch_types=[
        pltpu.VMEM((_PER_W,), jnp.int32),
        pltpu.VMEM((2, _PAIR, _D), jnp.float32),
        pltpu.SemaphoreType.DMA((2, 2)),
        pltpu.SemaphoreType.DMA((2,)),
    ],
)
def _emb_gather(idx_hbm, tab_hbm, out_hbm, idx_v, bufs, gsem, wsem):
    wid = lax.axis_index("s") * 2 + lax.axis_index("c")
    base = wid * _PER_W
    pltpu.sync_copy(idx_hbm.at[pl.ds(base, _PER_W)], idx_v)

    def _gather(off, p, h):
        pltpu.async_copy(tab_hbm.at[idx_v.at[pl.ds(off, _CHUNK)]],
                         bufs.at[p, pl.ds(h * _CHUNK, _CHUNK)], gsem.at[p, h])

    def _wait_gather(p, h):
        pltpu.make_async_copy(tab_hbm.at[idx_v.at[pl.ds(0, _CHUNK)]],
                              bufs.at[p, pl.ds(h * _CHUNK, _CHUNK)],
                              gsem.at[p, h]).wait()

    def _put(off, p):
        pltpu.async_copy(bufs.at[p], out_hbm.at[pl.ds(base + off, _PAIR)],
                         wsem.at[p])

    def _wait_put(p):
        pltpu.make_async_copy(bufs.at[p], out_hbm.at[pl.ds(base, _PAIR)],
                              wsem.at[p]).wait()

    _gather(0 * _CHUNK, 0, 0)
    _gather(1 * _CHUNK, 0, 1)
    _gather(2 * _CHUNK, 1, 0)
    _gather(3 * _CHUNK, 1, 1)
    _wait_gather(0, 0)
    _wait_gather(0, 1)
    _put(0, 0)

    def lap(j, carry):
        a = j * 4 * _CHUNK
        _wait_put(0)
        _gather(a, 0, 0)
        _gather(a + _CHUNK, 0, 1)
        _wait_gather(1, 0)
        _wait_gather(1, 1)
        _put(a - _PAIR, 1)
        _wait_put(1)
        _gather(a + 2 * _CHUNK, 1, 0)
        _gather(a + 3 * _CHUNK, 1, 1)
        _wait_gather(0, 0)
        _wait_gather(0, 1)
        _put(a, 0)
        return carry

    lax.fori_loop(1, _NLAP, lap, 0)

    last = (_NLAP - 1) * 4 * _CHUNK
    _wait_gather(1, 0)
    _wait_gather(1, 1)
    _put(last + _PAIR, 1)
    _wait_put(0)
    _wait_put(1)


def kernel(indices, weight):
    flat = indices.reshape(-1)
    out = _emb_gather(flat, weight)
    return out.reshape(indices.shape + (weight.shape[-1],))
